# Initial kernel scaffold; baseline (speedup 1.0000x reference)
import jax
import jax.numpy as jnp
from jax.experimental import pallas as pl


def kernel(cls_score_0, cls_score_1, cls_score_2, cls_score_3, cls_score_4, bbox_pred_0, bbox_pred_1, bbox_pred_2, bbox_pred_3, bbox_pred_4, shape_pred_0, shape_pred_1, shape_pred_2, shape_pred_3, shape_pred_4, loc_pred_0, loc_pred_1, loc_pred_2, loc_pred_3, loc_pred_4, x):
    raise NotImplementedError("write your pallas kernel here")


# speedup vs baseline: 1.0191x; 1.0191x over previous
import functools

import jax
import jax.numpy as jnp
import numpy as np
from jax.experimental import pallas as pl
from jax.experimental.pallas import tpu as pltpu

STRIDES = [8, 16, 32, 64, 128]
SIZES = [(128, 128), (64, 64), (32, 32), (16, 16), (8, 8)]
B = 4
NC = 80
TCOLS = 96
NANCH = sum(h * w for h, w in SIZES)
NPAD = 22528
NSEL = 1024
NMS_PRE = 1000
SCORE_THR = 0.05
IOU_THR = 0.5
MAX_PER_IMG = 100
MAXRATIO_G = 13.815511
MAXRATIO_P = float(abs(np.log(16.0 / 1000.0)))
LOC_THR = 0.01


def _k1_body(base, img_h, img_w, cls_ref, bbox_ref, shape_ref, loc_ref,
             px_ref, py_ref, maxsc_ref, tab_ref):
    cls = cls_ref[0]
    maxlogit = jnp.max(cls, axis=0)[None, :]
    loc_s = jax.nn.sigmoid(loc_ref[0])
    mask = loc_s >= LOC_THR
    ss = jax.nn.sigmoid(jax.nn.sigmoid(maxlogit))
    maxsc_ref[0] = jnp.where(mask, ss, 0.0)

    px = px_ref[...]
    py = py_ref[...]
    dw = jnp.clip(shape_ref[0, 0:1, :] * 0.14, -MAXRATIO_G, MAXRATIO_G)
    dh = jnp.clip(shape_ref[0, 1:2, :] * 0.14, -MAXRATIO_G, MAXRATIO_G)
    gw = base * jnp.exp(dw)
    gh = base * jnp.exp(dh)
    d2x = bbox_ref[0, 0:1, :]
    d2y = bbox_ref[0, 1:2, :]
    d2w = jnp.clip(bbox_ref[0, 2:3, :], -MAXRATIO_P, MAXRATIO_P)
    d2h = jnp.clip(bbox_ref[0, 3:4, :], -MAXRATIO_P, MAXRATIO_P)
    g2w = gw * jnp.exp(d2w)
    g2h = gh * jnp.exp(d2h)
    g2x = px + gw * d2x
    g2y = py + gh * d2y
    x1 = jnp.clip(g2x - 0.5 * g2w, 0.0, img_w)
    y1 = jnp.clip(g2y - 0.5 * g2h, 0.0, img_h)
    x2 = jnp.clip(g2x + 0.5 * g2w, 0.0, img_w)
    y2 = jnp.clip(g2y + 0.5 * g2h, 0.0, img_h)
    boxes = jnp.concatenate([x1, y1, x2, y2], axis=0)
    ch = cls.shape[1]
    row = jnp.concatenate(
        [jnp.transpose(boxes, (1, 0)),
         jnp.transpose(cls, (1, 0)),
         jnp.zeros((ch, TCOLS - 4 - NC), jnp.float32)], axis=1)
    tab_ref[0] = row


def _k1_level(lvl, cls, bbox, shp, loc, img_h, img_w):
    H, W = SIZES[lvl]
    hw = H * W
    ch = min(hw, 512)
    grid = (B, hw // ch)
    stride = STRIDES[lvl]
    xs = (np.arange(hw) % W).astype(np.float32) * stride
    ys = (np.arange(hw) // W).astype(np.float32) * stride
    px = jnp.asarray(xs)[None, :]
    py = jnp.asarray(ys)[None, :]
    base = float(stride * 4.0)
    out = pl.pallas_call(
        functools.partial(_k1_body, base, float(img_h), float(img_w)),
        grid=grid,
        in_specs=[
            pl.BlockSpec((1, NC, ch), lambda b, i: (b, 0, i)),
            pl.BlockSpec((1, 4, ch), lambda b, i: (b, 0, i)),
            pl.BlockSpec((1, 2, ch), lambda b, i: (b, 0, i)),
            pl.BlockSpec((1, 1, ch), lambda b, i: (b, 0, i)),
            pl.BlockSpec((1, ch), lambda b, i: (0, i)),
            pl.BlockSpec((1, ch), lambda b, i: (0, i)),
        ],
        out_specs=[
            pl.BlockSpec((1, 1, ch), lambda b, i: (b, 0, i)),
            pl.BlockSpec((1, ch, TCOLS), lambda b, i: (b, i, 0)),
        ],
        out_shape=[
            jax.ShapeDtypeStruct((B, 1, hw), jnp.float32),
            jax.ShapeDtypeStruct((B, hw, TCOLS), jnp.float32),
        ],
    )(cls.reshape(B, NC, hw), bbox.reshape(B, 4, hw),
      shp.reshape(B, 2, hw), loc.reshape(B, 1, hw), px, py)
    return out[0].reshape(B, hw), out[1]


def _nms_body(psc_ref, pcls_ref, bx1_ref, by1_ref, bx2_ref, by2_ref, out_ref):
    sc = psc_ref[0]
    clsv = pcls_ref[0]
    x1 = bx1_ref[0]
    y1 = by1_ref[0]
    x2 = bx2_ref[0]
    y2 = by2_ref[0]
    pos = jax.lax.broadcasted_iota(jnp.int32, (1, NSEL), 1)
    lane = jax.lax.broadcasted_iota(jnp.int32, (1, 128), 1)
    areas = jnp.clip(x2 - x1, 0.0, None) * jnp.clip(y2 - y1, 0.0, None)
    BIG = jnp.int32(NSEL + 7)

    def step(t, carry):
        validf, ax1, ay1, ax2, ay2, asc, acls, aok = carry
        valid = validf > 0.0
        j = jnp.min(jnp.where(valid, pos, BIG))
        ok = j < BIG
        oh = (pos == j).astype(jnp.float32)
        bx1 = jnp.sum(x1 * oh)
        by1 = jnp.sum(y1 * oh)
        bx2 = jnp.sum(x2 * oh)
        by2 = jnp.sum(y2 * oh)
        bsc = jnp.sum(sc * oh)
        bcl = jnp.sum(clsv * oh)
        barea = jnp.clip(bx2 - bx1, 0.0, None) * jnp.clip(by2 - by1, 0.0, None)
        ix1 = jnp.maximum(bx1, x1)
        iy1 = jnp.maximum(by1, y1)
        ix2 = jnp.minimum(bx2, x2)
        iy2 = jnp.minimum(by2, y2)
        inter = jnp.clip(ix2 - ix1, 0.0, None) * jnp.clip(iy2 - iy1, 0.0, None)
        iou = inter / (barea + areas - inter + 1e-6)
        suppress = (iou > IOU_THR) & (clsv == bcl) & ok
        valid = valid & jnp.logical_not(suppress) & (pos != j)
        validf = jnp.where(valid, 1.0, 0.0)
        okf = jnp.where(ok, 1.0, 0.0)
        loh = (lane == t).astype(jnp.float32)
        ax1 = ax1 + okf * bx1 * loh
        ay1 = ay1 + okf * by1 * loh
        ax2 = ax2 + okf * bx2 * loh
        ay2 = ay2 + okf * by2 * loh
        asc = asc + okf * bsc * loh
        acls = acls + jnp.where(ok, bcl, -1.0) * loh
        aok = aok + okf * loh
        return validf, ax1, ay1, ax2, ay2, asc, acls, aok

    z = jnp.zeros((1, 128), jnp.float32)
    valid0 = jnp.where(sc > 0.0, 1.0, 0.0)
    carry = (valid0, z, z, z, z, z, z, z)
    carry = jax.lax.fori_loop(0, MAX_PER_IMG, step, carry)
    _, ax1, ay1, ax2, ay2, asc, acls, aok = carry
    num = jnp.sum(aok) * jnp.ones((1, 128), jnp.float32)
    out_ref[0] = jnp.concatenate(
        [ax1, ay1, ax2, ay2, asc, acls, aok, num], axis=0)


def _nms_call(psc, pcls, bx1, by1, bx2, by2):
    spec = pl.BlockSpec((1, 1, NSEL), lambda b: (b, 0, 0))
    return pl.pallas_call(
        _nms_body,
        grid=(B,),
        in_specs=[spec] * 6,
        out_specs=pl.BlockSpec((1, 8, 128), lambda b: (b, 0, 0)),
        out_shape=jax.ShapeDtypeStruct((B, 8, 128), jnp.float32),
    )(*[a.reshape(B, 1, NSEL) for a in (psc, pcls, bx1, by1, bx2, by2)])


def kernel(cls_score_0, cls_score_1, cls_score_2, cls_score_3, cls_score_4,
           bbox_pred_0, bbox_pred_1, bbox_pred_2, bbox_pred_3, bbox_pred_4,
           shape_pred_0, shape_pred_1, shape_pred_2, shape_pred_3, shape_pred_4,
           loc_pred_0, loc_pred_1, loc_pred_2, loc_pred_3, loc_pred_4, x):
    img_h, img_w = x.shape[2], x.shape[3]
    cls_l = [cls_score_0, cls_score_1, cls_score_2, cls_score_3, cls_score_4]
    bbox_l = [bbox_pred_0, bbox_pred_1, bbox_pred_2, bbox_pred_3, bbox_pred_4]
    shp_l = [shape_pred_0, shape_pred_1, shape_pred_2, shape_pred_3, shape_pred_4]
    loc_l = [loc_pred_0, loc_pred_1, loc_pred_2, loc_pred_3, loc_pred_4]

    maxsc_l, tab_l = [], []
    for lvl in range(5):
        ms, tab = _k1_level(lvl, cls_l[lvl], bbox_l[lvl], shp_l[lvl],
                            loc_l[lvl], img_h, img_w)
        maxsc_l.append(ms)
        tab_l.append(tab)
    maxsc = jnp.concatenate(maxsc_l, axis=1)
    table = jnp.concatenate(tab_l, axis=1)

    csc, cidx = jax.lax.top_k(maxsc, NSEL)
    tsel = jnp.take_along_axis(table, cidx[..., None], axis=1)

    col = jnp.arange(TCOLS)[None, None, :]
    prow = jnp.arange(NSEL)[None, :, None]
    valid = ((col >= 4) & (col < 4 + NC) & (prow < NMS_PRE)
             & (csc[..., None] > 0.0))
    ssv = jax.nn.sigmoid(jax.nn.sigmoid(tsel))
    pair = jnp.where(valid & (ssv >= SCORE_THR), ssv, -1.0)
    pair = pair.reshape(B, NSEL * TCOLS)

    psc, pidx = jax.lax.top_k(pair, NSEL)
    psc = jnp.where(jnp.arange(NSEL)[None, :] < NMS_PRE, psc, -1.0)
    candpos = pidx // TCOLS
    pcls = (pidx % TCOLS - 4).astype(jnp.float32)
    pbox = jnp.take_along_axis(tsel[:, :, 0:4], candpos[..., None], axis=1)

    out = _nms_call(psc, pcls, pbox[:, :, 0], pbox[:, :, 1],
                    pbox[:, :, 2], pbox[:, :, 3])
    bx = jnp.stack([out[:, 0, :MAX_PER_IMG], out[:, 1, :MAX_PER_IMG],
                    out[:, 2, :MAX_PER_IMG], out[:, 3, :MAX_PER_IMG]], axis=-1)
    scores = out[:, 4, :MAX_PER_IMG]
    cls_id = out[:, 5, :MAX_PER_IMG].astype(jnp.int32)
    num = out[:, 7, 0].astype(jnp.int32)
    return (num, bx, scores, cls_id)
